# Initial kernel scaffold; baseline (speedup 1.0000x reference)
import jax
import jax.numpy as jnp
from jax.experimental import pallas as pl


def kernel(boxes, classification):
    raise NotImplementedError("write your pallas kernel here")


# speedup vs baseline: 12.9366x; 12.9366x over previous
import functools

import jax
import jax.numpy as jnp
from jax.experimental import pallas as pl
from jax.experimental.pallas import tpu as pltpu

_B, _N, _C = 8, 20000, 80
_NMS_T = 0.5
_SCORE_T = 0.05
_MAXDET = 100
_R, _L = 160, 128
_NPAD = _R * _L
_NEG_INF = float("-inf")

_INTERPRET = False


def _scores_kernel(cls_ref, s_ref, l_ref):
    x = cls_ref[0]
    m = jnp.max(x, axis=-1)
    cio = jax.lax.broadcasted_iota(jnp.int32, x.shape, 1)
    lab = jnp.min(jnp.where(x == m[:, None], cio, _C), axis=-1)
    s_ref[0, 0] = m
    l_ref[0, 0] = lab


def _nms_kernel(s_ref, b_ref, l_ref, os_ref, ox1_ref, oy1_ref, ox2_ref,
                oy2_ref, ol_ref):
    scores = s_ref[...]
    x1 = b_ref[:, 0]
    y1 = b_ref[:, 1]
    x2 = b_ref[:, 2]
    y2 = b_ref[:, 3]
    labs = l_ref[...]

    area = jnp.maximum(x2 - x1, 0.0) * jnp.maximum(y2 - y1, 0.0)
    lin = (jax.lax.broadcasted_iota(jnp.int32, (_B, _R, _L), 1) * _L
           + jax.lax.broadcasted_iota(jnp.int32, (_B, _R, _L), 2))
    lane = jax.lax.broadcasted_iota(jnp.int32, (_B, 1, _L), 2)

    work0 = jnp.where(scores > _SCORE_T, scores, _NEG_INF)
    zf = jnp.full((_B, 1, _L), -1.0, dtype=jnp.float32)
    zi = jnp.full((_B, 1, _L), -1, dtype=jnp.int32)

    def body(k, carry):
        work, o_s, o_x1, o_y1, o_x2, o_y2, o_l = carry
        m = jnp.max(work, axis=(1, 2), keepdims=True)
        tied = work == m
        idx = jnp.min(jnp.where(tied, lin, _NPAD), axis=(1, 2),
                      keepdims=True)
        onehot = lin == idx
        bx1 = jnp.sum(jnp.where(onehot, x1, 0.0), axis=(1, 2), keepdims=True)
        by1 = jnp.sum(jnp.where(onehot, y1, 0.0), axis=(1, 2), keepdims=True)
        bx2 = jnp.sum(jnp.where(onehot, x2, 0.0), axis=(1, 2), keepdims=True)
        by2 = jnp.sum(jnp.where(onehot, y2, 0.0), axis=(1, 2), keepdims=True)
        blab = jnp.max(jnp.where(onehot, labs, 0), axis=(1, 2), keepdims=True)

        ix1 = jnp.maximum(bx1, x1)
        iy1 = jnp.maximum(by1, y1)
        ix2 = jnp.minimum(bx2, x2)
        iy2 = jnp.minimum(by2, y2)
        inter = jnp.maximum(ix2 - ix1, 0.0) * jnp.maximum(iy2 - iy1, 0.0)
        a1 = jnp.maximum(bx2 - bx1, 0.0) * jnp.maximum(by2 - by1, 0.0)
        iou = inter / (a1 + area - inter + 1e-8)
        sup = (iou > _NMS_T) | onehot
        work = jnp.where(sup, _NEG_INF, work)

        valid = m > _NEG_INF
        hit = lane == k
        o_s = jnp.where(hit, jnp.where(valid, m, -1.0), o_s)
        o_x1 = jnp.where(hit, jnp.where(valid, bx1, -1.0), o_x1)
        o_y1 = jnp.where(hit, jnp.where(valid, by1, -1.0), o_y1)
        o_x2 = jnp.where(hit, jnp.where(valid, bx2, -1.0), o_x2)
        o_y2 = jnp.where(hit, jnp.where(valid, by2, -1.0), o_y2)
        o_l = jnp.where(hit, jnp.where(valid, blab, -1), o_l)
        return work, o_s, o_x1, o_y1, o_x2, o_y2, o_l

    carry = (work0, zf, zf, zf, zf, zf, zi)
    _, o_s, o_x1, o_y1, o_x2, o_y2, o_l = jax.lax.fori_loop(
        0, _MAXDET, body, carry)
    os_ref[...] = o_s
    ox1_ref[...] = o_x1
    oy1_ref[...] = o_y1
    ox2_ref[...] = o_x2
    oy2_ref[...] = o_y2
    ol_ref[...] = o_l


@jax.jit
def kernel(boxes, classification):
    scores, labels = pl.pallas_call(
        _scores_kernel,
        grid=(_B,),
        in_specs=[pl.BlockSpec((1, _N, _C), lambda b: (b, 0, 0))],
        out_specs=[
            pl.BlockSpec((1, 1, _N), lambda b: (b, 0, 0)),
            pl.BlockSpec((1, 1, _N), lambda b: (b, 0, 0)),
        ],
        out_shape=[
            jax.ShapeDtypeStruct((_B, 1, _N), jnp.float32),
            jax.ShapeDtypeStruct((_B, 1, _N), jnp.int32),
        ],
        interpret=_INTERPRET,
    )(classification)

    scores = scores.reshape(_B, _N)
    labels = labels.reshape(_B, _N)

    pad = _NPAD - _N
    s_p = jnp.pad(scores, ((0, 0), (0, pad)),
                  constant_values=_NEG_INF).reshape(_B, _R, _L)
    l_p = jnp.pad(labels, ((0, 0), (0, pad))).reshape(_B, _R, _L)
    b_p = jnp.pad(jnp.moveaxis(boxes, 2, 1), ((0, 0), (0, 0), (0, pad)))
    b_p = b_p.reshape(_B, 4, _R, _L)

    outs = pl.pallas_call(
        _nms_kernel,
        grid=(1,),
        in_specs=[
            pl.BlockSpec((_B, _R, _L), lambda i: (0, 0, 0)),
            pl.BlockSpec((_B, 4, _R, _L), lambda i: (0, 0, 0, 0)),
            pl.BlockSpec((_B, _R, _L), lambda i: (0, 0, 0)),
        ],
        out_specs=[pl.BlockSpec((_B, 1, _L), lambda i: (0, 0, 0))] * 6,
        out_shape=[jax.ShapeDtypeStruct((_B, 1, _L), jnp.float32)] * 5
        + [jax.ShapeDtypeStruct((_B, 1, _L), jnp.int32)],
        interpret=_INTERPRET,
    )(s_p, b_p, l_p)
    o_s, o_x1, o_y1, o_x2, o_y2, o_l = outs

    out_scores = o_s[:, 0, :_MAXDET]
    out_labels = o_l[:, 0, :_MAXDET]
    out_boxes = jnp.stack(
        [o_x1[:, 0, :_MAXDET], o_y1[:, 0, :_MAXDET],
         o_x2[:, 0, :_MAXDET], o_y2[:, 0, :_MAXDET]], axis=-1)
    return out_boxes, out_scores, out_labels
